# baseline (speedup 1.0000x reference)
import functools

import jax
import jax.numpy as jnp
from jax import lax
from jax.experimental import pallas as pl
from jax.experimental.pallas import tpu as pltpu
from jax.experimental.pallas import tpu_sc as plsc

HIDDEN = 64
PADH = 128
BLK = 128
NW = 32
L = 16


def _gather_kernel(seq: int, batch: int, nrows: int):
    mesh = plsc.VectorSubcoreMesh(core_axis_name="c", subcore_axis_name="s")

    @functools.partial(
        pl.kernel,
        mesh=mesh,
        out_type=jax.ShapeDtypeStruct((seq, HIDDEN, batch), jnp.float32),
        scratch_types=[
            pltpu.VMEM((seq, BLK), jnp.int32),
            pltpu.VMEM((BLK, PADH), jnp.float32),
            pltpu.VMEM((BLK, PADH), jnp.float32),
            pltpu.VMEM((HIDDEN, BLK), jnp.float32),
            pltpu.VMEM((HIDDEN, BLK), jnp.float32),
            pltpu.SemaphoreType.DMA,
            pltpu.SemaphoreType.DMA,
            pltpu.SemaphoreType.DMA,
        ],
        compiler_params=pltpu.CompilerParams(
            use_tc_tiling_on_sc=True, needs_layout_passes=False),
    )
    def k(ids_hbm, table_hbm, out_hbm, idx_v, g0, g1, t0, t1, sg0, sg1, st):
        wid = lax.axis_index("s") * 2 + lax.axis_index("c")
        i0 = wid * BLK
        gbanks = (g0, g1)
        tbanks = (t0, t1)
        gsems = (sg0, sg1)

        pltpu.sync_copy(ids_hbm.at[:, pl.ds(i0, BLK)], idx_v)

        def fire_gather(j, p):
            pltpu.async_copy(table_hbm.at[idx_v.at[j]], gbanks[p], gsems[p])

        def drain_gather(p):
            pltpu.make_async_copy(
                table_hbm.at[idx_v.at[0]], gbanks[p], gsems[p]).wait()

        def transpose_block(p):
            g, t = gbanks[p], tbanks[p]
            for h in range(HIDDEN):
                for c in range(BLK // L):
                    rows = lax.iota(jnp.int32, L) + (c * L)
                    cols = jnp.full((L,), h, jnp.int32)
                    t[h, pl.ds(c * L, L)] = plsc.load_gather(g, [rows, cols])

        def fire_out(j, p):
            pltpu.async_copy(
                tbanks[p], out_hbm.at[j, :, pl.ds(i0, BLK)], st)

        def drain_out(p):
            pltpu.make_async_copy(
                tbanks[p], out_hbm.at[0, :, pl.ds(i0, BLK)], st).wait()

        fire_gather(0, 0)

        def body(jj, carry):
            for p in range(2):
                j = 2 * jj + p
                drain_gather(p)

                @pl.when(j + 1 < seq)
                def _():
                    fire_gather(j + 1, 1 - p)

                @pl.when(j >= 2)
                def _():
                    drain_out(p)

                transpose_block(p)
                fire_out(j, p)
            return carry

        lax.fori_loop(0, seq // 2, body, 0)
        drain_out(0)
        drain_out(1)

    return k


def kernel(ids, table):
    b, s, _ = ids.shape
    idx_t = jnp.transpose(ids[:, :, 0]).astype(jnp.int32)
    table_p = jnp.pad(table, ((0, 0), (0, PADH - HIDDEN)))
    out_p = _gather_kernel(s, b, table.shape[0])(idx_t, table_p)
    return jnp.transpose(out_p, (2, 0, 1))
